# baseline (speedup 1.0000x reference)
import functools

import jax
import jax.numpy as jnp
from jax import lax
from jax.experimental import pallas as pl
from jax.experimental.pallas import tpu as pltpu
from jax.experimental.pallas import tpu_sc as plsc

_SEQ = 4096
_K = 4
_M = 64
_D = 1024
_LANES = 16
_SC_CORES = 1


def _build_w_sc(tts_flat):
    info = plsc.get_sparse_core_info()
    num_cores = _SC_CORES
    nw = num_cores * info.num_subcores
    toks_per_w = _SEQ // nw
    w_len = toks_per_w * _M

    mesh = plsc.VectorSubcoreMesh(
        core_axis_name="c", subcore_axis_name="s", num_cores=num_cores
    )

    chunk = _K * toks_per_w

    @functools.partial(
        pl.kernel,
        mesh=mesh,
        out_type=jax.ShapeDtypeStruct((_SEQ, _M), jnp.float32),
        scratch_types=[
            pltpu.VMEM((chunk,), jnp.int32),
            pltpu.VMEM((toks_per_w, _M), jnp.float32),
        ],
        compiler_params=pltpu.CompilerParams(needs_layout_passes=False),
    )
    def build_w(tts_hbm, w_hbm, idx_v, w_v):
        wid = lax.axis_index("c") * info.num_subcores + lax.axis_index("s")
        base_t = wid * toks_per_w
        pltpu.sync_copy(tts_hbm.at[pl.ds(wid * chunk, chunk)], idx_v)

        zeros16 = jnp.zeros((_LANES,), jnp.float32)

        def zero_body(i, carry):
            for r in range(4):
                for u in range(_M // _LANES):
                    w_v[i * 4 + r, pl.ds(u * _LANES, _LANES)] = zeros16
            return carry

        lax.fori_loop(0, toks_per_w // 4, zero_body, 0)

        lane = lax.broadcasted_iota(jnp.int32, (_LANES,), 0)
        quarter = jnp.full((_LANES,), 0.25, jnp.float32)

        for g in range(toks_per_w // _LANES):
            t16 = lane + g * _LANES
            for j in range(_K):
                col = idx_v[pl.ds(j * toks_per_w + g * _LANES, _LANES)]
                plsc.addupdate_scatter(w_v, [t16, col], quarter)

        pltpu.sync_copy(w_v, w_hbm.at[pl.ds(base_t, toks_per_w), :])

    return build_w(tts_flat)


def _mix_body(w_ref, ss_ref, out_ref):
    out_ref[0] = jnp.dot(
        w_ref[...], ss_ref[0], preferred_element_type=jnp.float32
    )


def _mix_tc(w, set_states):
    bs = 2048
    return pl.pallas_call(
        _mix_body,
        grid=(set_states.shape[0], _SEQ // bs),
        in_specs=[
            pl.BlockSpec((bs, _M), lambda b, s: (s, 0)),
            pl.BlockSpec((1, _M, _D), lambda b, s: (b, 0, 0)),
        ],
        out_specs=pl.BlockSpec((1, bs, _D), lambda b, s: (b, s, 0)),
        out_shape=jax.ShapeDtypeStruct(
            (set_states.shape[0], _SEQ, _D), jnp.float32
        ),
    )(w, set_states)


def kernel(set_states, token_to_sets):
    info = plsc.get_sparse_core_info()
    nw = _SC_CORES * info.num_subcores
    tts_flat = (
        token_to_sets.astype(jnp.int32)
        .reshape(nw, _SEQ // nw, _K)
        .transpose(0, 2, 1)
        .reshape(-1)
    )
    w = _build_w_sc(tts_flat)
    return _mix_tc(w, set_states)


# speedup vs baseline: 1.0145x; 1.0145x over previous
import functools

import jax
import jax.numpy as jnp
from jax import lax
from jax.experimental import pallas as pl
from jax.experimental.pallas import tpu as pltpu
from jax.experimental.pallas import tpu_sc as plsc

_SEQ = 4096
_K = 4
_M = 64
_D = 1024
_LANES = 16
_SC_CORES = 2


def _build_w_sc(tts_flat):
    info = plsc.get_sparse_core_info()
    num_cores = _SC_CORES
    nw = num_cores * info.num_subcores
    toks_per_w = _SEQ // nw
    w_len = toks_per_w * _M

    mesh = plsc.VectorSubcoreMesh(
        core_axis_name="c", subcore_axis_name="s", num_cores=num_cores
    )

    chunk = _K * toks_per_w

    @functools.partial(
        pl.kernel,
        mesh=mesh,
        out_type=jax.ShapeDtypeStruct((_SEQ, _M), jnp.float32),
        scratch_types=[
            pltpu.VMEM((chunk,), jnp.int32),
            pltpu.VMEM((toks_per_w, _M), jnp.float32),
            pltpu.SemaphoreType.DMA,
        ],
        compiler_params=pltpu.CompilerParams(needs_layout_passes=False),
    )
    def build_w(tts_hbm, w_hbm, idx_v, w_v, sem):
        wid = lax.axis_index("c") * info.num_subcores + lax.axis_index("s")
        base_t = wid * toks_per_w
        cp = pltpu.async_copy(
            tts_hbm.at[pl.ds(wid * chunk, chunk)], idx_v, sem
        )

        zeros16 = jnp.zeros((_LANES,), jnp.float32)

        def zero_body(i, carry):
            for r in range(4):
                for u in range(_M // _LANES):
                    w_v[i * 4 + r, pl.ds(u * _LANES, _LANES)] = zeros16
            return carry

        lax.fori_loop(0, toks_per_w // 4, zero_body, 0)
        cp.wait()

        lane = lax.broadcasted_iota(jnp.int32, (_LANES,), 0)
        quarter = jnp.full((_LANES,), 0.25, jnp.float32)

        for g in range(toks_per_w // _LANES):
            t16 = lane + g * _LANES
            for j in range(_K):
                col = idx_v[pl.ds(j * toks_per_w + g * _LANES, _LANES)]
                plsc.addupdate_scatter(w_v, [t16, col], quarter)

        pltpu.sync_copy(w_v, w_hbm.at[pl.ds(base_t, toks_per_w), :])

    return build_w(tts_flat)


def _mix_body(w_ref, ss_ref, out_ref):
    out_ref[0] = jnp.dot(
        w_ref[...], ss_ref[0], preferred_element_type=jnp.float32
    )


def _mix_tc(w, set_states):
    bs = 2048
    return pl.pallas_call(
        _mix_body,
        grid=(set_states.shape[0], _SEQ // bs),
        in_specs=[
            pl.BlockSpec((bs, _M), lambda b, s: (s, 0)),
            pl.BlockSpec((1, _M, _D), lambda b, s: (b, 0, 0)),
        ],
        out_specs=pl.BlockSpec((1, bs, _D), lambda b, s: (b, s, 0)),
        out_shape=jax.ShapeDtypeStruct(
            (set_states.shape[0], _SEQ, _D), jnp.float32
        ),
    )(w, set_states)


def kernel(set_states, token_to_sets):
    info = plsc.get_sparse_core_info()
    nw = _SC_CORES * info.num_subcores
    tts_flat = (
        token_to_sets.astype(jnp.int32)
        .reshape(nw, _SEQ // nw, _K)
        .transpose(0, 2, 1)
        .reshape(-1)
    )
    w = _build_w_sc(tts_flat)
    return _mix_tc(w, set_states)
